# Initial kernel scaffold; baseline (speedup 1.0000x reference)
import functools

import jax
import jax.numpy as jnp
from jax import lax
from jax.experimental import pallas as pl
from jax.experimental.pallas import tpu as pltpu
from jax.experimental.pallas import tpu_sc as plsc

N_NODES = 10000
N_EDGES = 160000
IN_DIM = 256
OUT_DIM = 256

NC = 2
NS = 16
HALF = IN_DIM // 2
E_PER_TILE = N_EDGES // NS
CH = 80
NCHUNK = E_PER_TILE // CH
ROWS_PER_TILE = N_NODES // NS


def _sc_segment_sum(xs, gidx, dstc, zeros):
  mesh = plsc.VectorSubcoreMesh(core_axis_name="c", subcore_axis_name="s")

  @functools.partial(
      pl.kernel,
      out_type=(
          jax.ShapeDtypeStruct((N_NODES, HALF), jnp.float32),
          jax.ShapeDtypeStruct((N_NODES, HALF), jnp.float32),
      ),
      mesh=mesh,
      scratch_types=[
          pltpu.VMEM_SHARED((N_NODES, HALF), jnp.float32),
          pltpu.VMEM((NCHUNK, CH), jnp.int32),
          pltpu.VMEM((NCHUNK, CH), jnp.int32),
          pltpu.VMEM((CH, HALF), jnp.float32),
          pltpu.SemaphoreType.DMA,
      ],
  )
  def k(xs_hbm, gidx_hbm, dstc_hbm, zeros_hbm, agg0_hbm, agg1_hbm,
        aggs, idxv, dstv, rows, gsem):
    c = lax.axis_index("c")
    s = lax.axis_index("s")
    base = s * ROWS_PER_TILE

    pltpu.sync_copy(zeros_hbm, aggs.at[pl.ds(base, ROWS_PER_TILE)])
    pltpu.sync_copy(gidx_hbm.at[c, s], idxv)
    pltpu.sync_copy(dstc_hbm.at[s], dstv)
    plsc.subcore_barrier()

    def chunk(i, carry):
      pltpu.async_copy(xs_hbm.at[idxv.at[i]], rows, gsem).wait()
      pltpu.sync_copy(rows, aggs.at[dstv.at[i]], add=True)
      return carry

    lax.fori_loop(0, NCHUNK, chunk, 0)
    plsc.subcore_barrier()

    @pl.when(c == 0)
    def _():
      pltpu.sync_copy(aggs.at[pl.ds(base, ROWS_PER_TILE)],
                      agg0_hbm.at[pl.ds(base, ROWS_PER_TILE)])

    @pl.when(c == 1)
    def _():
      pltpu.sync_copy(aggs.at[pl.ds(base, ROWS_PER_TILE)],
                      agg1_hbm.at[pl.ds(base, ROWS_PER_TILE)])

  return k(xs, gidx, dstc, zeros)


def _mm_body(a0, a1, xr, w0, w1, wr, b, o):
  acc = jnp.dot(a0[...], w0[...], preferred_element_type=jnp.float32)
  acc += jnp.dot(a1[...], w1[...], preferred_element_type=jnp.float32)
  acc += jnp.dot(xr[...], wr[...], preferred_element_type=jnp.float32)
  o[...] = acc + b[...]


def _tc_linear(agg0, agg1, x, W_rel, b_rel, W_root):
  BM = 1000
  grid = (N_NODES // BM,)
  return pl.pallas_call(
      _mm_body,
      grid=grid,
      in_specs=[
          pl.BlockSpec((BM, HALF), lambda i: (i, 0)),
          pl.BlockSpec((BM, HALF), lambda i: (i, 0)),
          pl.BlockSpec((BM, IN_DIM), lambda i: (i, 0)),
          pl.BlockSpec((HALF, OUT_DIM), lambda i: (0, 0)),
          pl.BlockSpec((HALF, OUT_DIM), lambda i: (0, 0)),
          pl.BlockSpec((IN_DIM, OUT_DIM), lambda i: (0, 0)),
          pl.BlockSpec((1, OUT_DIM), lambda i: (0, 0)),
      ],
      out_specs=pl.BlockSpec((BM, OUT_DIM), lambda i: (i, 0)),
      out_shape=jax.ShapeDtypeStruct((N_NODES, OUT_DIM), jnp.float32),
  )(agg0, agg1, x, W_rel[:HALF], W_rel[HALF:], W_root, b_rel.reshape(1, -1))


@jax.jit
def kernel(x, edge_index, W_rel, b_rel, W_root):
  src = edge_index[0].astype(jnp.int32)
  dst = edge_index[1].astype(jnp.int32)

  xs = jnp.concatenate([x[:, :HALF], x[:, HALF:]], axis=0)
  gidx = jnp.stack([src, src + N_NODES]).reshape(NC, NS, NCHUNK, CH)
  dstc = dst.reshape(NS, NCHUNK, CH)
  zeros = jnp.zeros((ROWS_PER_TILE, HALF), jnp.float32)

  agg0, agg1 = _sc_segment_sum(xs, gidx, dstc, zeros)
  return _tc_linear(agg0, agg1, x, W_rel, b_rel, W_root)


# speedup vs baseline: 4.7176x; 4.7176x over previous
import functools

import jax
import jax.numpy as jnp
from jax import lax
from jax.experimental import pallas as pl
from jax.experimental.pallas import tpu as pltpu
from jax.experimental.pallas import tpu_sc as plsc

N_NODES = 10000
N_EDGES = 160000
IN_DIM = 256
OUT_DIM = 256

NC = 2
NS = 16
HALF = IN_DIM // 2
E_PER_TILE = N_EDGES // NS
CH = 80
NCHUNK = E_PER_TILE // CH
NPAD = 10240
ROWS_PER_TILE = NPAD // NS


def _sc_segment_sum(xs, gidx, dstc, zeros):
  mesh = plsc.VectorSubcoreMesh(core_axis_name="c", subcore_axis_name="s")

  @functools.partial(
      pl.kernel,
      out_type=(
          jax.ShapeDtypeStruct((NPAD, HALF), jnp.float32),
          jax.ShapeDtypeStruct((NPAD, HALF), jnp.float32),
      ),
      mesh=mesh,
      scratch_types=[
          pltpu.VMEM_SHARED((NPAD, HALF), jnp.float32),
          pltpu.VMEM((NCHUNK, CH), jnp.int32),
          pltpu.VMEM((NCHUNK, CH), jnp.int32),
          pltpu.VMEM((CH, HALF), jnp.float32),
          pltpu.SemaphoreType.DMA,
      ],
  )
  def k(xs_hbm, gidx_hbm, dstc_hbm, zeros_hbm, agg0_hbm, agg1_hbm,
        aggs, idxv, dstv, rows, gsem):
    c = lax.axis_index("c")
    s = lax.axis_index("s")
    base = s * ROWS_PER_TILE

    pltpu.sync_copy(zeros_hbm, aggs.at[pl.ds(base, ROWS_PER_TILE)])
    pltpu.sync_copy(gidx_hbm.at[c, s], idxv)
    pltpu.sync_copy(dstc_hbm.at[s], dstv)
    plsc.subcore_barrier()

    def chunk(i, carry):
      pltpu.async_copy(xs_hbm.at[idxv.at[i]], rows, gsem).wait()
      pltpu.sync_copy(rows, aggs.at[dstv.at[i]], add=True)
      return carry

    lax.fori_loop(0, NCHUNK, chunk, 0)
    plsc.subcore_barrier()

    @pl.when(c == 0)
    def _():
      pltpu.sync_copy(aggs.at[pl.ds(base, ROWS_PER_TILE)],
                      agg0_hbm.at[pl.ds(base, ROWS_PER_TILE)])

    @pl.when(c == 1)
    def _():
      pltpu.sync_copy(aggs.at[pl.ds(base, ROWS_PER_TILE)],
                      agg1_hbm.at[pl.ds(base, ROWS_PER_TILE)])

  return k(xs, gidx, dstc, zeros)


def _mm_body(a0, a1, xr, w0, w1, wr, b, o):
  acc = jnp.dot(a0[...], w0[...], preferred_element_type=jnp.float32)
  acc += jnp.dot(a1[...], w1[...], preferred_element_type=jnp.float32)
  acc += jnp.dot(xr[...], wr[...], preferred_element_type=jnp.float32)
  o[...] = acc + b[...]


def _tc_linear(agg0, agg1, x, W_rel, b_rel, W_root):
  BM = 1000
  grid = (N_NODES // BM,)
  return pl.pallas_call(
      _mm_body,
      grid=grid,
      in_specs=[
          pl.BlockSpec((BM, HALF), lambda i: (i, 0)),
          pl.BlockSpec((BM, HALF), lambda i: (i, 0)),
          pl.BlockSpec((BM, IN_DIM), lambda i: (i, 0)),
          pl.BlockSpec((HALF, OUT_DIM), lambda i: (0, 0)),
          pl.BlockSpec((HALF, OUT_DIM), lambda i: (0, 0)),
          pl.BlockSpec((IN_DIM, OUT_DIM), lambda i: (0, 0)),
          pl.BlockSpec((1, OUT_DIM), lambda i: (0, 0)),
      ],
      out_specs=pl.BlockSpec((BM, OUT_DIM), lambda i: (i, 0)),
      out_shape=jax.ShapeDtypeStruct((N_NODES, OUT_DIM), jnp.float32),
  )(agg0, agg1, x, W_rel[:HALF], W_rel[HALF:], W_root, b_rel.reshape(1, -1))


@jax.jit
def kernel(x, edge_index, W_rel, b_rel, W_root):
  src = edge_index[0].astype(jnp.int32)
  dst = edge_index[1].astype(jnp.int32)

  xs = jnp.concatenate([x[:, :HALF], x[:, HALF:]], axis=0)
  gidx = jnp.stack([src, src + N_NODES]).reshape(NC, NS, NCHUNK, CH)
  dstc = dst.reshape(NS, NCHUNK, CH)
  zeros = jnp.zeros((ROWS_PER_TILE, HALF), jnp.float32)

  agg0, agg1 = _sc_segment_sum(xs, gidx, dstc, zeros)
  return _tc_linear(agg0[:N_NODES], agg1[:N_NODES], x, W_rel, b_rel, W_root)
